# Initial kernel scaffold; baseline (speedup 1.0000x reference)
import jax
import jax.numpy as jnp
from jax.experimental import pallas as pl


def kernel(word_ids, edge_index, edge_weight, edge_time, graph_ids, y_data, word_embeds, adapt_W, adapt_b, bn_gamma, bn_beta, temp_W, temp_b, gcn_W1, gcn_b1, gcn_W2, gcn_b2, out_W, out_b):
    raise NotImplementedError("write your pallas kernel here")


# speedup vs baseline: 4.8075x; 4.8075x over previous
import functools

import jax
import jax.numpy as jnp
from jax import lax
from jax.experimental import pallas as pl
from jax.experimental.pallas import tpu as pltpu
from jax.experimental.pallas import tpu_sc as plsc

N = 10000
E = 320000
B = 32
F = 128
T = 8
VOCAB = 15000
EPS = 1e-5

NP = 10240
NE = 12288
K = 128
EP = E + 1088
MDUMP = N
NW = 32
RPW = NP // 16
BLK = 512
MAXCH = EP // (32 * 128) + 1
FBLK = 128

_mesh = plsc.VectorSubcoreMesh(core_axis_name="c", subcore_axis_name="s")


def _lane_select(ref16, lane):
    return ref16[pl.ds(0, 16)][lane]


@functools.partial(
    pl.kernel,
    out_type=jax.ShapeDtypeStruct((NE, F), jnp.float32),
    mesh=_mesh,
    scratch_types=[
        pltpu.VMEM((384,), jnp.int32),
        pltpu.VMEM((384, F), jnp.float32),
        pltpu.SemaphoreType.DMA,
    ],
)
def _sc_embed(tbl_ref, wid_ref, out_ref, idx_v, rows_v, sem):
    c = lax.axis_index("c")
    s = lax.axis_index("s")
    wid = s * 2 + c
    base = pl.multiple_of(wid * 384, 128)
    pltpu.sync_copy(wid_ref.at[pl.ds(base, 384)], idx_v)
    for j in range(3):
        pltpu.async_copy(
            tbl_ref.at[idx_v.at[pl.ds(j * 128, 128)]],
            rows_v.at[pl.ds(j * 128, 128)],
            sem,
        ).wait()
    pltpu.sync_copy(rows_v, out_ref.at[pl.ds(base, 384)])


@functools.partial(
    pl.kernel,
    out_type=jax.ShapeDtypeStruct((2, NP, F), jnp.float32),
    mesh=_mesh,
    scratch_types=[
        pltpu.VMEM((16,), jnp.int32),
        pltpu.VMEM((K,), jnp.int32),
        pltpu.VMEM((K,), jnp.int32),
        pltpu.VMEM((K, F), jnp.float32),
        pltpu.VMEM_SHARED((NP, F), jnp.float32),
        pltpu.SemaphoreType.DMA,
    ],
)
def _sc_masks(src_ref, dst_ref, oe_ref, z_ref, pat_ref, out_ref,
              oe_v, src_v, dst_v, pat_v, acc, sem):
    c = lax.axis_index("c")
    s = lax.axis_index("s")
    wid = s * 2 + c
    pltpu.sync_copy(oe_ref, oe_v)
    sr = pl.multiple_of(s * RPW, 128)
    pltpu.sync_copy(z_ref.at[pl.ds(sr, RPW)], acc.at[pl.ds(sr, RPW)])
    plsc.subcore_barrier()
    for t in range(T):
        st = oe_v[pl.ds(0, 16)][t]
        en = oe_v[pl.ds(0, 16)][t + 8]
        pltpu.sync_copy(pat_ref.at[pl.ds(t * K, K)], pat_v)

        def chunk(j, carry, st=st, en=en):
            e = st + wid * K + j * (NW * K)

            @pl.when(e < en)
            def _():
                ea = pl.multiple_of(e, 8)
                pltpu.sync_copy(src_ref.at[pl.ds(ea, K)], src_v)
                pltpu.sync_copy(dst_ref.at[pl.ds(ea, K)], dst_v)
                li = lax.broadcasted_iota(jnp.int32, (16,), 0)
                for b in range(8):
                    valid = (e + b * 16 + li) < en
                    sl = pl.ds(b * 16, 16)
                    src_v[sl] = jnp.where(valid, src_v[sl], MDUMP)
                    dst_v[sl] = jnp.where(valid, dst_v[sl], MDUMP)
                pltpu.sync_copy(pat_v, acc.at[src_v], add=True)
                pltpu.sync_copy(pat_v, acc.at[dst_v], add=True)

            return carry

        lax.fori_loop(0, MAXCH, chunk, 0)
    plsc.subcore_barrier()
    pltpu.sync_copy(acc.at[pl.ds(sr, RPW)], out_ref.at[c].at[pl.ds(sr, RPW)])


@functools.partial(
    pl.kernel,
    out_type=jax.ShapeDtypeStruct((2, NP, F), jnp.float32),
    mesh=_mesh,
    scratch_types=[
        pltpu.VMEM((16,), jnp.int32),
        pltpu.VMEM((K,), jnp.int32),
        pltpu.VMEM((K,), jnp.int32),
        pltpu.VMEM((K,), jnp.float32),
        pltpu.VMEM((K, F), jnp.float32),
        pltpu.VMEM_SHARED((NP, F), jnp.float32),
        pltpu.SemaphoreType.DMA,
    ],
)
def _sc_msg(g_ref, src_ref, dst_ref, ew_ref, bnd_ref, z_ref, out_ref,
            bnd_v, idx_v, dst_v, ew_v, rows_v, acc, sem):
    c = lax.axis_index("c")
    s = lax.axis_index("s")
    wid = s * 2 + c
    pltpu.sync_copy(bnd_ref, bnd_v)
    start = _lane_select(bnd_v, 0)
    end = _lane_select(bnd_v, 1)
    sr = pl.multiple_of(s * RPW, 128)
    pltpu.sync_copy(z_ref.at[pl.ds(sr, RPW)], acc.at[pl.ds(sr, RPW)])
    plsc.subcore_barrier()

    def chunk(j, carry):
        e = start + wid * K + j * (NW * K)

        @pl.when(e < end)
        def _():
            ea = pl.multiple_of(e, 8)
            pltpu.sync_copy(src_ref.at[pl.ds(ea, K)], idx_v)
            pltpu.sync_copy(dst_ref.at[pl.ds(ea, K)], dst_v)
            pltpu.sync_copy(ew_ref.at[pl.ds(ea, K)], ew_v)
            pltpu.async_copy(g_ref.at[idx_v], rows_v, sem).wait()

            def scale(g2, carry2):
                ew16 = ew_v[pl.ds(g2 * 16, 16)]
                for l in range(16):
                    wv = jnp.full((16,), ew16[l], jnp.float32)
                    k = g2 * 16 + l
                    for jj in range(8):
                        rows_v[k, pl.ds(jj * 16, 16)] = (
                            rows_v[k, pl.ds(jj * 16, 16)] * wv)
                return carry2

            lax.fori_loop(0, K // 16, scale, 0)
            pltpu.sync_copy(rows_v, acc.at[dst_v], add=True)

        return carry

    lax.fori_loop(0, MAXCH, chunk, 0)
    plsc.subcore_barrier()
    pltpu.sync_copy(acc.at[pl.ds(sr, RPW)],
                    out_ref.at[c].at[pl.ds(sr, RPW)])


def _row_spec(blk):
    return pl.BlockSpec((blk, F), lambda i: (i, 0))


def _col_spec(blk):
    return pl.BlockSpec((blk, 1), lambda i: (i, 0))


def _fix(shape):
    return pl.BlockSpec(shape, lambda i: tuple(0 for _ in shape))


def _tc_linear_body(x_ref, w_ref, b_ref, o_ref):
    o_ref[...] = (
        jnp.dot(x_ref[...], w_ref[...], preferred_element_type=jnp.float32)
        + b_ref[...]
    )


def _tc_linear(x, w, b2d):
    return pl.pallas_call(
        _tc_linear_body,
        grid=(NP // BLK,),
        in_specs=[_row_spec(BLK), _fix((F, F)), _fix((1, F))],
        out_specs=_row_spec(BLK),
        out_shape=jax.ShapeDtypeStruct((NP, F), jnp.float32),
    )(x, w, b2d)


def _tc_stats_body(h_ref, h0_ref, c0_ref, c1_ref, o_ref):
    i = pl.program_id(0)

    @pl.when(i == 0)
    def _():
        o_ref[...] = jnp.zeros((8, 256), jnp.float32)

    rowid = (lax.broadcasted_iota(jnp.int32, (BLK, 1), 0) + i * BLK)
    m = (((c0_ref[...] + c1_ref[...]) > 0.0) & (rowid < N)).astype(jnp.float32)
    hh = jnp.concatenate([h_ref[...], h0_ref[...]], axis=1)
    mh = hh * m
    o_ref[0:1, :] += jnp.sum(mh, axis=0, keepdims=True)
    o_ref[1:2, :] += jnp.sum(mh * hh, axis=0, keepdims=True)
    o_ref[2:3, :] += jnp.sum(m) * jnp.ones((1, 256), jnp.float32)


def _tc_stats(h, h0, c0, c1):
    return pl.pallas_call(
        _tc_stats_body,
        grid=(NP // BLK,),
        in_specs=[_row_spec(BLK), _row_spec(BLK), _col_spec(BLK), _col_spec(BLK)],
        out_specs=_fix((8, 256)),
        out_shape=jax.ShapeDtypeStruct((8, 256), jnp.float32),
    )(h, h0, c0, c1)


def _tc_bnmm_body(h_ref, h0_ref, st_ref, g_ref, be_ref, tw_ref, tb_ref, w1_ref,
                  o_ref):
    cnt = jnp.maximum(st_ref[2:3, :], 1.0)
    mean = st_ref[0:1, :] / cnt
    var = jnp.maximum(st_ref[1:2, :] / cnt - mean * mean, 0.0)
    g = g_ref[...]
    g2 = g * g
    scale = g2 * lax.rsqrt(g2 * var + EPS * (var + EPS))
    shift = be_ref[...] - scale * mean
    hh = jnp.concatenate([h_ref[...], h0_ref[...]], axis=1)
    z = hh * scale + shift
    hs = jnp.tanh(
        jnp.dot(z, tw_ref[...], preferred_element_type=jnp.float32) + tb_ref[...]
    )
    o_ref[...] = jnp.dot(hs, w1_ref[...], preferred_element_type=jnp.float32)


def _tc_bnmm(h, h0, stats, gam, bet, tw, tb, w1):
    return pl.pallas_call(
        _tc_bnmm_body,
        grid=(NP // BLK,),
        in_specs=[
            _row_spec(BLK), _row_spec(BLK), _fix((8, 256)), _fix((1, 256)),
            _fix((1, 256)), _fix((256, F)), _fix((1, F)), _fix((F, F)),
        ],
        out_specs=_row_spec(BLK),
        out_shape=jax.ShapeDtypeStruct((NP, F), jnp.float32),
    )(h, h0, stats, gam, bet, tw, tb, w1)


def _tc_relumm_body(p0_ref, p1_ref, b_ref, w_ref, o_ref):
    x = jnp.maximum(p0_ref[...] + p1_ref[...] + b_ref[...], 0.0)
    o_ref[...] = jnp.dot(x, w_ref[...], preferred_element_type=jnp.float32)


def _tc_relumm(p0, p1, b2d, w):
    return pl.pallas_call(
        _tc_relumm_body,
        grid=(NP // BLK,),
        in_specs=[_row_spec(BLK), _row_spec(BLK), _fix((1, F)), _fix((F, F))],
        out_specs=_row_spec(BLK),
        out_shape=jax.ShapeDtypeStruct((NP, F), jnp.float32),
    )(p0, p1, b2d, w)


def _tc_update_body(q0_ref, q1_ref, b_ref, h_ref, c0_ref, c1_ref, o_ref):
    m = (c0_ref[...] + c1_ref[...]) > 0.0
    hs = jnp.maximum(q0_ref[...] + q1_ref[...] + b_ref[...], 0.0)
    o_ref[...] = jnp.where(m, hs, h_ref[...])


def _tc_update(q0, q1, b2d, h, c0, c1):
    return pl.pallas_call(
        _tc_update_body,
        grid=(NP // BLK,),
        in_specs=[
            _row_spec(BLK), _row_spec(BLK), _fix((1, F)), _row_spec(BLK),
            _col_spec(BLK), _col_spec(BLK),
        ],
        out_specs=_row_spec(BLK),
        out_shape=jax.ShapeDtypeStruct((NP, F), jnp.float32),
    )(q0, q1, b2d, h, c0, c1)


def _tc_final_body(h_ref, gid_ref, y_ref, w_ref, b_ref, yp_ref, ls_ref, pooled):
    i = pl.program_id(0)
    nb = pl.num_programs(0)

    @pl.when(i == 0)
    def _():
        pooled[...] = jnp.full((B, F), -1e30, jnp.float32)

    gid = gid_ref[...]
    hb = h_ref[...]
    for b in range(B):
        mb = gid == b
        vb = jnp.where(mb, hb, -1e30)
        pooled[b:b + 1, :] = jnp.maximum(
            pooled[b:b + 1, :], jnp.max(vb, axis=0, keepdims=True)
        )

    @pl.when(i == nb - 1)
    def _():
        logits = jnp.sum(pooled[...] * w_ref[...], axis=1, keepdims=True) + b_ref[0, 0]
        y = y_ref[...]
        ll = (jnp.maximum(logits, 0.0) - logits * y
              + jnp.log1p(jnp.exp(-jnp.abs(logits))))
        ls_ref[...] = jnp.mean(ll) * jnp.ones((8, 128), jnp.float32)
        yp_ref[...] = jnp.broadcast_to(jax.nn.sigmoid(logits), (B, F))


def _tc_final(h, gid2d, y2d, w2d, b2d):
    return pl.pallas_call(
        _tc_final_body,
        grid=(NP // FBLK,),
        in_specs=[
            _row_spec(FBLK), _col_spec(FBLK), _fix((B, 1)), _fix((1, F)),
            _fix((1, 1)),
        ],
        out_specs=[_fix((B, F)), _fix((8, 128))],
        out_shape=[
            jax.ShapeDtypeStruct((B, F), jnp.float32),
            jax.ShapeDtypeStruct((8, 128), jnp.float32),
        ],
        scratch_shapes=[pltpu.VMEM((B, F), jnp.float32)],
    )(h, gid2d, y2d, w2d, b2d)


def kernel(word_ids, edge_index, edge_weight, edge_time, graph_ids, y_data,
           word_embeds, adapt_W, adapt_b, bn_gamma, bn_beta, temp_W, temp_b,
           gcn_W1, gcn_b1, gcn_W2, gcn_b2, out_W, out_b):
    i32 = jnp.int32
    f32 = jnp.float32
    wi = jnp.concatenate(
        [word_ids.astype(i32), jnp.zeros((NE - N,), i32)])
    src = edge_index[0].astype(i32)
    dst = edge_index[1].astype(i32)
    et = edge_time.astype(i32)
    ew = edge_weight.astype(f32)

    order = jnp.argsort(et, stable=True)
    ts = et[order]
    counts = jnp.bincount(et, length=T).astype(i32)
    cnt_pad = ((counts + 7) // 8) * 8 + K
    offs = jnp.concatenate([jnp.zeros((1,), i32), jnp.cumsum(cnt_pad)])[:T]
    bst = jnp.concatenate([jnp.zeros((1,), i32), jnp.cumsum(counts)])[:T]
    pos = offs[ts] + (jnp.arange(E, dtype=i32) - bst[ts])
    src_s = jnp.zeros((EP,), i32).at[pos].set(src[order])
    dst_s = jnp.zeros((EP,), i32).at[pos].set(dst[order])
    ew_s = jnp.zeros((EP,), f32).at[pos].set(ew[order])
    ends = offs + counts
    offends = jnp.concatenate([offs, ends])

    zeros_nf = jnp.zeros((NP, F), f32)
    gid2d = jnp.concatenate(
        [graph_ids.astype(i32), jnp.full((NP - N,), B, i32)]).reshape(NP, 1)

    gam = bn_gamma.astype(f32).reshape(1, 256)
    bet = bn_beta.astype(f32).reshape(1, 256)
    tb2 = temp_b.astype(f32).reshape(1, F)
    ab2 = adapt_b.astype(f32).reshape(1, F)
    b12 = gcn_b1.astype(f32).reshape(1, F)
    b22 = gcn_b2.astype(f32).reshape(1, F)
    y2d = y_data.astype(f32).reshape(B, 1)
    w2d = out_W.astype(f32).reshape(1, F)
    ob2 = out_b.astype(f32).reshape(1, 1)

    h0raw = _sc_embed(word_embeds.astype(f32), wi)
    h0 = _tc_linear(h0raw[:NP], adapt_W.astype(f32), ab2)

    pat = (jnp.arange(F, dtype=i32)[None, :] // 16
           == jnp.arange(T, dtype=i32)[:, None]).astype(f32)
    pat = jnp.broadcast_to(pat[:, None, :], (T, K, F)).reshape(T * K, F)
    cnt_out = _sc_masks(src_s, dst_s, offends, zeros_nf, pat)
    cntr = cnt_out[0] + cnt_out[1]

    h = h0
    zc = jnp.zeros((NP, 1), f32)
    for t in range(T):
        c0 = cntr[:, 16 * t:16 * t + 1]
        c1 = zc
        stats = _tc_stats(h, h0, c0, c1)
        g1 = _tc_bnmm(h, h0, stats, gam, bet, temp_W.astype(f32), tb2,
                      gcn_W1.astype(f32))
        bounds = jnp.concatenate(
            [offs[t:t + 1], ends[t:t + 1], jnp.zeros((14,), i32)])
        p = _sc_msg(g1, src_s, dst_s, ew_s, bounds, zeros_nf)
        g2 = _tc_relumm(p[0], p[1], b12, gcn_W2.astype(f32))
        q = _sc_msg(g2, src_s, dst_s, ew_s, bounds, zeros_nf)
        h = _tc_update(q[0], q[1], b22, h, c0, c1)

    yp, ls = _tc_final(h, gid2d, y2d, w2d, ob2)
    return ls[0, 0], yp[:, :1]


# speedup vs baseline: 5.2043x; 1.0825x over previous
import functools

import jax
import jax.numpy as jnp
from jax import lax
from jax.experimental import pallas as pl
from jax.experimental.pallas import tpu as pltpu
from jax.experimental.pallas import tpu_sc as plsc

N = 10000
E = 320000
B = 32
F = 128
T = 8
VOCAB = 15000
EPS = 1e-5

NP = 10240
NE = 12288
K = 128
EP = E + 1088
MDUMP = N
NW = 32
RPW = NP // 16
BLK = 512
MAXCH = EP // (32 * 128) + 1
FBLK = 128

_mesh = plsc.VectorSubcoreMesh(core_axis_name="c", subcore_axis_name="s")


def _lane_select(ref16, lane):
    return ref16[pl.ds(0, 16)][lane]


@functools.partial(
    pl.kernel,
    out_type=jax.ShapeDtypeStruct((NE, F), jnp.float32),
    mesh=_mesh,
    scratch_types=[
        pltpu.VMEM((384,), jnp.int32),
        pltpu.VMEM((384, F), jnp.float32),
        pltpu.SemaphoreType.DMA,
    ],
)
def _sc_embed(tbl_ref, wid_ref, out_ref, idx_v, rows_v, sem):
    c = lax.axis_index("c")
    s = lax.axis_index("s")
    wid = s * 2 + c
    base = pl.multiple_of(wid * 384, 128)
    pltpu.sync_copy(wid_ref.at[pl.ds(base, 384)], idx_v)
    for j in range(3):
        pltpu.async_copy(
            tbl_ref.at[idx_v.at[pl.ds(j * 128, 128)]],
            rows_v.at[pl.ds(j * 128, 128)],
            sem,
        ).wait()
    pltpu.sync_copy(rows_v, out_ref.at[pl.ds(base, 384)])


@functools.partial(
    pl.kernel,
    out_type=jax.ShapeDtypeStruct((2, NP, F), jnp.float32),
    mesh=_mesh,
    scratch_types=[
        pltpu.VMEM((16,), jnp.int32),
        pltpu.VMEM((K,), jnp.int32),
        pltpu.VMEM((K,), jnp.int32),
        pltpu.VMEM((K, F), jnp.float32),
        pltpu.VMEM_SHARED((NP, F), jnp.float32),
        pltpu.SemaphoreType.DMA,
    ],
)
def _sc_masks(src_ref, dst_ref, oe_ref, z_ref, pat_ref, out_ref,
              oe_v, src_v, dst_v, pat_v, acc, sem):
    c = lax.axis_index("c")
    s = lax.axis_index("s")
    wid = s * 2 + c
    pltpu.sync_copy(oe_ref, oe_v)
    sr = pl.multiple_of(s * RPW, 128)
    pltpu.sync_copy(z_ref.at[pl.ds(sr, RPW)], acc.at[pl.ds(sr, RPW)])
    plsc.subcore_barrier()
    for t in range(T):
        st = oe_v[pl.ds(0, 16)][t]
        en = oe_v[pl.ds(0, 16)][t + 8]
        pltpu.sync_copy(pat_ref.at[pl.ds(t * K, K)], pat_v)

        def chunk(j, carry, st=st, en=en):
            e = st + wid * K + j * (NW * K)

            @pl.when(e < en)
            def _():
                ea = pl.multiple_of(e, 8)
                pltpu.sync_copy(src_ref.at[pl.ds(ea, K)], src_v)
                pltpu.sync_copy(dst_ref.at[pl.ds(ea, K)], dst_v)
                li = lax.broadcasted_iota(jnp.int32, (16,), 0)
                for b in range(8):
                    valid = (e + b * 16 + li) < en
                    sl = pl.ds(b * 16, 16)
                    src_v[sl] = jnp.where(valid, src_v[sl], MDUMP)
                    dst_v[sl] = jnp.where(valid, dst_v[sl], MDUMP)
                pltpu.sync_copy(pat_v, acc.at[src_v], add=True)
                pltpu.sync_copy(pat_v, acc.at[dst_v], add=True)

            return carry

        lax.fori_loop(0, MAXCH, chunk, 0)
    plsc.subcore_barrier()
    pltpu.sync_copy(acc.at[pl.ds(sr, RPW)], out_ref.at[c].at[pl.ds(sr, RPW)])


@functools.partial(
    pl.kernel,
    out_type=jax.ShapeDtypeStruct((2, NP, F), jnp.float32),
    mesh=_mesh,
    scratch_types=[
        pltpu.VMEM((16,), jnp.int32),
        pltpu.VMEM((K,), jnp.int32),
        pltpu.VMEM((K,), jnp.int32),
        pltpu.VMEM((K,), jnp.float32),
        pltpu.VMEM((K, F), jnp.float32),
        pltpu.VMEM_SHARED((NP, F), jnp.float32),
        pltpu.SemaphoreType.DMA,
    ],
)
def _sc_msg(g_ref, src_ref, dst_ref, ew_ref, bnd_ref, z_ref, out_ref,
            bnd_v, idx_v, dst_v, ew_v, rows_v, acc, sem):
    c = lax.axis_index("c")
    s = lax.axis_index("s")
    wid = s * 2 + c
    pltpu.sync_copy(bnd_ref, bnd_v)
    start = _lane_select(bnd_v, 0)
    end = _lane_select(bnd_v, 1)
    sr = pl.multiple_of(s * RPW, 128)
    pltpu.sync_copy(z_ref.at[pl.ds(sr, RPW)], acc.at[pl.ds(sr, RPW)])
    plsc.subcore_barrier()

    def chunk(j, carry):
        e = start + wid * K + j * (NW * K)

        @pl.when(e < end)
        def _():
            ea = pl.multiple_of(e, 8)
            pltpu.sync_copy(src_ref.at[pl.ds(ea, K)], idx_v)
            pltpu.sync_copy(dst_ref.at[pl.ds(ea, K)], dst_v)
            pltpu.sync_copy(ew_ref.at[pl.ds(ea, K)], ew_v)
            pltpu.async_copy(g_ref.at[idx_v], rows_v, sem).wait()

            def scale(g2, carry2):
                ew16 = ew_v[pl.ds(g2 * 16, 16)]
                for l in range(16):
                    wv = jnp.full((16,), ew16[l], jnp.float32)
                    k = g2 * 16 + l
                    for jj in range(8):
                        rows_v[k, pl.ds(jj * 16, 16)] = (
                            rows_v[k, pl.ds(jj * 16, 16)] * wv)
                return carry2

            lax.fori_loop(0, K // 16, scale, 0)
            pltpu.sync_copy(rows_v, acc.at[dst_v], add=True)

        return carry

    lax.fori_loop(0, MAXCH, chunk, 0)
    plsc.subcore_barrier()
    pltpu.sync_copy(acc.at[pl.ds(sr, RPW)],
                    out_ref.at[c].at[pl.ds(sr, RPW)])


def _row_spec(blk):
    return pl.BlockSpec((blk, F), lambda i: (i, 0))


def _col_spec(blk):
    return pl.BlockSpec((blk, 1), lambda i: (i, 0))


def _fix(shape):
    return pl.BlockSpec(shape, lambda i: tuple(0 for _ in shape))


def _tc_linear_body(x_ref, w_ref, b_ref, o_ref):
    o_ref[...] = (
        jnp.dot(x_ref[...], w_ref[...], preferred_element_type=jnp.float32)
        + b_ref[...]
    )


def _tc_linear(x, w, b2d):
    return pl.pallas_call(
        _tc_linear_body,
        grid=(NP // BLK,),
        in_specs=[_row_spec(BLK), _fix((F, F)), _fix((1, F))],
        out_specs=_row_spec(BLK),
        out_shape=jax.ShapeDtypeStruct((NP, F), jnp.float32),
    )(x, w, b2d)


def _tc_stats_body(h_ref, h0_ref, c0_ref, c1_ref, o_ref):
    i = pl.program_id(0)

    @pl.when(i == 0)
    def _():
        o_ref[...] = jnp.zeros((8, 256), jnp.float32)

    rowid = (lax.broadcasted_iota(jnp.int32, (BLK, 1), 0) + i * BLK)
    m = (((c0_ref[...] + c1_ref[...]) > 0.0) & (rowid < N)).astype(jnp.float32)
    hh = jnp.concatenate([h_ref[...], h0_ref[...]], axis=1)
    mh = hh * m
    o_ref[0:1, :] += jnp.sum(mh, axis=0, keepdims=True)
    o_ref[1:2, :] += jnp.sum(mh * hh, axis=0, keepdims=True)
    o_ref[2:3, :] += jnp.sum(m) * jnp.ones((1, 256), jnp.float32)


def _tc_stats(h, h0, c0, c1):
    return pl.pallas_call(
        _tc_stats_body,
        grid=(NP // BLK,),
        in_specs=[_row_spec(BLK), _row_spec(BLK), _col_spec(BLK), _col_spec(BLK)],
        out_specs=_fix((8, 256)),
        out_shape=jax.ShapeDtypeStruct((8, 256), jnp.float32),
    )(h, h0, c0, c1)


def _tc_bnmm_body(h_ref, h0_ref, st_ref, g_ref, be_ref, tw_ref, tb_ref, w1_ref,
                  o_ref):
    cnt = jnp.maximum(st_ref[2:3, :], 1.0)
    mean = st_ref[0:1, :] / cnt
    var = jnp.maximum(st_ref[1:2, :] / cnt - mean * mean, 0.0)
    g = g_ref[...]
    g2 = g * g
    scale = g2 * lax.rsqrt(g2 * var + EPS * (var + EPS))
    shift = be_ref[...] - scale * mean
    hh = jnp.concatenate([h_ref[...], h0_ref[...]], axis=1)
    z = hh * scale + shift
    hs = jnp.tanh(
        jnp.dot(z, tw_ref[...], preferred_element_type=jnp.float32) + tb_ref[...]
    )
    o_ref[...] = jnp.dot(hs, w1_ref[...], preferred_element_type=jnp.float32)


def _tc_bnmm(h, h0, stats, gam, bet, tw, tb, w1):
    return pl.pallas_call(
        _tc_bnmm_body,
        grid=(NP // BLK,),
        in_specs=[
            _row_spec(BLK), _row_spec(BLK), _fix((8, 256)), _fix((1, 256)),
            _fix((1, 256)), _fix((256, F)), _fix((1, F)), _fix((F, F)),
        ],
        out_specs=_row_spec(BLK),
        out_shape=jax.ShapeDtypeStruct((NP, F), jnp.float32),
    )(h, h0, stats, gam, bet, tw, tb, w1)


def _tc_relumm_body(p0_ref, p1_ref, b_ref, w_ref, o_ref):
    x = jnp.maximum(p0_ref[...] + p1_ref[...] + b_ref[...], 0.0)
    o_ref[...] = jnp.dot(x, w_ref[...], preferred_element_type=jnp.float32)


def _tc_relumm(p0, p1, b2d, w):
    return pl.pallas_call(
        _tc_relumm_body,
        grid=(NP // BLK,),
        in_specs=[_row_spec(BLK), _row_spec(BLK), _fix((1, F)), _fix((F, F))],
        out_specs=_row_spec(BLK),
        out_shape=jax.ShapeDtypeStruct((NP, F), jnp.float32),
    )(p0, p1, b2d, w)


def _tc_update_body(q0_ref, q1_ref, b_ref, h_ref, c0_ref, c1_ref, o_ref):
    m = (c0_ref[...] + c1_ref[...]) > 0.0
    hs = jnp.maximum(q0_ref[...] + q1_ref[...] + b_ref[...], 0.0)
    o_ref[...] = jnp.where(m, hs, h_ref[...])


def _tc_update(q0, q1, b2d, h, c0, c1):
    return pl.pallas_call(
        _tc_update_body,
        grid=(NP // BLK,),
        in_specs=[
            _row_spec(BLK), _row_spec(BLK), _fix((1, F)), _row_spec(BLK),
            _col_spec(BLK), _col_spec(BLK),
        ],
        out_specs=_row_spec(BLK),
        out_shape=jax.ShapeDtypeStruct((NP, F), jnp.float32),
    )(q0, q1, b2d, h, c0, c1)


def _tc_final_body(h_ref, gid_ref, y_ref, w_ref, b_ref, yp_ref, ls_ref, pooled):
    i = pl.program_id(0)
    nb = pl.num_programs(0)

    @pl.when(i == 0)
    def _():
        pooled[...] = jnp.full((B, F), -1e30, jnp.float32)

    gid = gid_ref[...]
    hb = h_ref[...]
    for b in range(B):
        mb = gid == b
        vb = jnp.where(mb, hb, -1e30)
        pooled[b:b + 1, :] = jnp.maximum(
            pooled[b:b + 1, :], jnp.max(vb, axis=0, keepdims=True)
        )

    @pl.when(i == nb - 1)
    def _():
        logits = jnp.sum(pooled[...] * w_ref[...], axis=1, keepdims=True) + b_ref[0, 0]
        y = y_ref[...]
        ll = (jnp.maximum(logits, 0.0) - logits * y
              + jnp.log1p(jnp.exp(-jnp.abs(logits))))
        ls_ref[...] = jnp.mean(ll) * jnp.ones((8, 128), jnp.float32)
        yp_ref[...] = jnp.broadcast_to(jax.nn.sigmoid(logits), (B, F))


def _tc_final(h, gid2d, y2d, w2d, b2d):
    return pl.pallas_call(
        _tc_final_body,
        grid=(NP // FBLK,),
        in_specs=[
            _row_spec(FBLK), _col_spec(FBLK), _fix((B, 1)), _fix((1, F)),
            _fix((1, 1)),
        ],
        out_specs=[_fix((B, F)), _fix((8, 128))],
        out_shape=[
            jax.ShapeDtypeStruct((B, F), jnp.float32),
            jax.ShapeDtypeStruct((8, 128), jnp.float32),
        ],
        scratch_shapes=[pltpu.VMEM((B, F), jnp.float32)],
    )(h, gid2d, y2d, w2d, b2d)


def kernel(word_ids, edge_index, edge_weight, edge_time, graph_ids, y_data,
           word_embeds, adapt_W, adapt_b, bn_gamma, bn_beta, temp_W, temp_b,
           gcn_W1, gcn_b1, gcn_W2, gcn_b2, out_W, out_b):
    i32 = jnp.int32
    f32 = jnp.float32
    wi = jnp.concatenate(
        [word_ids.astype(i32), jnp.zeros((NE - N,), i32)])
    src = edge_index[0].astype(i32)
    dst = edge_index[1].astype(i32)
    et = edge_time.astype(i32)
    ew = edge_weight.astype(f32)

    onehot = (et[:, None] == jnp.arange(T, dtype=i32)[None, :]).astype(i32)
    cums = jnp.cumsum(onehot, axis=0)
    counts = cums[-1]
    rank = jnp.sum(cums * onehot, axis=1) - 1
    cnt_pad = ((counts + 7) // 8) * 8 + K
    offs = jnp.concatenate([jnp.zeros((1,), i32), jnp.cumsum(cnt_pad)])[:T]
    pos = offs[et] + rank
    src_s = jnp.zeros((EP,), i32).at[pos].set(src)
    dst_s = jnp.zeros((EP,), i32).at[pos].set(dst)
    ew_s = jnp.zeros((EP,), f32).at[pos].set(ew)
    ends = offs + counts
    offends = jnp.concatenate([offs, ends])

    zeros_nf = jnp.zeros((NP, F), f32)
    gid2d = jnp.concatenate(
        [graph_ids.astype(i32), jnp.full((NP - N,), B, i32)]).reshape(NP, 1)

    gam = bn_gamma.astype(f32).reshape(1, 256)
    bet = bn_beta.astype(f32).reshape(1, 256)
    tb2 = temp_b.astype(f32).reshape(1, F)
    ab2 = adapt_b.astype(f32).reshape(1, F)
    b12 = gcn_b1.astype(f32).reshape(1, F)
    b22 = gcn_b2.astype(f32).reshape(1, F)
    y2d = y_data.astype(f32).reshape(B, 1)
    w2d = out_W.astype(f32).reshape(1, F)
    ob2 = out_b.astype(f32).reshape(1, 1)

    h0raw = _sc_embed(word_embeds.astype(f32), wi)
    h0 = _tc_linear(h0raw[:NP], adapt_W.astype(f32), ab2)

    pat = (jnp.arange(F, dtype=i32)[None, :] // 16
           == jnp.arange(T, dtype=i32)[:, None]).astype(f32)
    pat = jnp.broadcast_to(pat[:, None, :], (T, K, F)).reshape(T * K, F)
    cnt_out = _sc_masks(src_s, dst_s, offends, zeros_nf, pat)
    cntr = cnt_out[0] + cnt_out[1]

    h = h0
    zc = jnp.zeros((NP, 1), f32)
    for t in range(T):
        c0 = cntr[:, 16 * t:16 * t + 1]
        c1 = zc
        stats = _tc_stats(h, h0, c0, c1)
        g1 = _tc_bnmm(h, h0, stats, gam, bet, temp_W.astype(f32), tb2,
                      gcn_W1.astype(f32))
        bounds = jnp.concatenate(
            [offs[t:t + 1], ends[t:t + 1], jnp.zeros((14,), i32)])
        p = _sc_msg(g1, src_s, dst_s, ew_s, bounds, zeros_nf)
        g2 = _tc_relumm(p[0], p[1], b12, gcn_W2.astype(f32))
        q = _sc_msg(g2, src_s, dst_s, ew_s, bounds, zeros_nf)
        h = _tc_update(q[0], q[1], b22, h, c0, c1)

    yp, ls = _tc_final(h, gid2d, y2d, w2d, ob2)
    return ls[0, 0], yp[:, :1]


# speedup vs baseline: 5.3088x; 1.0201x over previous
import functools

import jax
import jax.numpy as jnp
from jax import lax
from jax.experimental import pallas as pl
from jax.experimental.pallas import tpu as pltpu
from jax.experimental.pallas import tpu_sc as plsc

N = 10000
E = 320000
B = 32
F = 128
T = 8
VOCAB = 15000
EPS = 1e-5

NP = 10240
NE = 12288
K = 128
EP = E + 1088
MDUMP = N
NW = 32
RPW = NP // 16
BLK = 512
MAXCH = EP // (32 * 128) + 1
FBLK = 128

_mesh = plsc.VectorSubcoreMesh(core_axis_name="c", subcore_axis_name="s")


def _lane_select(ref16, lane):
    return ref16[pl.ds(0, 16)][lane]


@functools.partial(
    pl.kernel,
    out_type=jax.ShapeDtypeStruct((NE, F), jnp.float32),
    mesh=_mesh,
    scratch_types=[
        pltpu.VMEM((384,), jnp.int32),
        pltpu.VMEM((384, F), jnp.float32),
        pltpu.SemaphoreType.DMA,
    ],
)
def _sc_embed(tbl_ref, wid_ref, out_ref, idx_v, rows_v, sem):
    c = lax.axis_index("c")
    s = lax.axis_index("s")
    wid = s * 2 + c
    base = pl.multiple_of(wid * 384, 128)
    pltpu.sync_copy(wid_ref.at[pl.ds(base, 384)], idx_v)
    for j in range(3):
        pltpu.async_copy(
            tbl_ref.at[idx_v.at[pl.ds(j * 128, 128)]],
            rows_v.at[pl.ds(j * 128, 128)],
            sem,
        ).wait()
    pltpu.sync_copy(rows_v, out_ref.at[pl.ds(base, 384)])


@functools.partial(
    pl.kernel,
    out_type=jax.ShapeDtypeStruct((2, NP, F), jnp.float32),
    mesh=_mesh,
    scratch_types=[
        pltpu.VMEM((16,), jnp.int32),
        pltpu.VMEM((K,), jnp.int32),
        pltpu.VMEM((K,), jnp.int32),
        pltpu.VMEM((K, F), jnp.float32),
        pltpu.VMEM_SHARED((NP, F), jnp.float32),
        pltpu.SemaphoreType.DMA,
    ],
)
def _sc_masks(src_ref, dst_ref, oe_ref, z_ref, pat_ref, out_ref,
              oe_v, src_v, dst_v, pat_v, acc, sem):
    c = lax.axis_index("c")
    s = lax.axis_index("s")
    wid = s * 2 + c
    pltpu.sync_copy(oe_ref, oe_v)
    sr = pl.multiple_of(s * RPW, 128)
    pltpu.sync_copy(z_ref.at[pl.ds(sr, RPW)], acc.at[pl.ds(sr, RPW)])
    plsc.subcore_barrier()
    for t in range(T):
        st = oe_v[pl.ds(0, 16)][t]
        en = oe_v[pl.ds(0, 16)][t + 8]
        pltpu.sync_copy(pat_ref.at[pl.ds(t * K, K)], pat_v)

        def chunk(j, carry, st=st, en=en):
            e = st + wid * K + j * (NW * K)

            @pl.when(e < en)
            def _():
                ea = pl.multiple_of(e, 8)
                pltpu.sync_copy(src_ref.at[pl.ds(ea, K)], src_v)
                pltpu.sync_copy(dst_ref.at[pl.ds(ea, K)], dst_v)
                li = lax.broadcasted_iota(jnp.int32, (16,), 0)
                for b in range(8):
                    valid = (e + b * 16 + li) < en
                    sl = pl.ds(b * 16, 16)
                    src_v[sl] = jnp.where(valid, src_v[sl], MDUMP)
                    dst_v[sl] = jnp.where(valid, dst_v[sl], MDUMP)
                pltpu.sync_copy(pat_v, acc.at[src_v], add=True)
                pltpu.sync_copy(pat_v, acc.at[dst_v], add=True)

            return carry

        lax.fori_loop(0, MAXCH, chunk, 0)
    plsc.subcore_barrier()
    pltpu.sync_copy(acc.at[pl.ds(sr, RPW)], out_ref.at[c].at[pl.ds(sr, RPW)])


@functools.partial(
    pl.kernel,
    out_type=jax.ShapeDtypeStruct((2, NP, F), jnp.float32),
    mesh=_mesh,
    scratch_types=[
        pltpu.VMEM((16,), jnp.int32),
        pltpu.VMEM((K,), jnp.int32),
        pltpu.VMEM((K,), jnp.int32),
        pltpu.VMEM((K,), jnp.float32),
        pltpu.VMEM((K, F), jnp.float32),
        pltpu.VMEM_SHARED((NP, F), jnp.float32),
        pltpu.SemaphoreType.DMA,
    ],
)
def _sc_msg(g_ref, src_ref, dst_ref, ew_ref, bnd_ref, z_ref, out_ref,
            bnd_v, idx_v, dst_v, ew_v, rows_v, acc, sem):
    c = lax.axis_index("c")
    s = lax.axis_index("s")
    wid = s * 2 + c
    pltpu.sync_copy(bnd_ref, bnd_v)
    start = _lane_select(bnd_v, 0)
    end = _lane_select(bnd_v, 1)
    sr = pl.multiple_of(s * RPW, 128)
    pltpu.sync_copy(z_ref.at[pl.ds(sr, RPW)], acc.at[pl.ds(sr, RPW)])
    plsc.subcore_barrier()

    def chunk(j, carry):
        e = start + wid * K + j * (NW * K)

        @pl.when(e < end)
        def _():
            ea = pl.multiple_of(e, 8)
            pltpu.sync_copy(src_ref.at[pl.ds(ea, K)], idx_v)
            pltpu.sync_copy(dst_ref.at[pl.ds(ea, K)], dst_v)
            pltpu.sync_copy(ew_ref.at[pl.ds(ea, K)], ew_v)
            pltpu.async_copy(g_ref.at[idx_v], rows_v, sem).wait()

            def scale(g2, carry2):
                ew16 = ew_v[pl.ds(g2 * 16, 16)]
                for l in range(16):
                    wv = jnp.full((16,), ew16[l], jnp.float32)
                    k = g2 * 16 + l
                    for jj in range(8):
                        rows_v[k, pl.ds(jj * 16, 16)] = (
                            rows_v[k, pl.ds(jj * 16, 16)] * wv)
                return carry2

            lax.fori_loop(0, K // 16, scale, 0)
            pltpu.sync_copy(rows_v, acc.at[dst_v], add=True)

        return carry

    lax.fori_loop(0, MAXCH, chunk, 0)
    plsc.subcore_barrier()
    pltpu.sync_copy(acc.at[pl.ds(sr, RPW)],
                    out_ref.at[c].at[pl.ds(sr, RPW)])


def _row_spec(blk):
    return pl.BlockSpec((blk, F), lambda i: (i, 0))


def _col_spec(blk):
    return pl.BlockSpec((blk, 1), lambda i: (i, 0))


def _fix(shape):
    return pl.BlockSpec(shape, lambda i: tuple(0 for _ in shape))


def _tc_linear_body(x_ref, w_ref, b_ref, o_ref):
    o_ref[...] = (
        jnp.dot(x_ref[...], w_ref[...], preferred_element_type=jnp.float32)
        + b_ref[...]
    )


def _tc_linear(x, w, b2d):
    return pl.pallas_call(
        _tc_linear_body,
        grid=(NP // BLK,),
        in_specs=[_row_spec(BLK), _fix((F, F)), _fix((1, F))],
        out_specs=_row_spec(BLK),
        out_shape=jax.ShapeDtypeStruct((NP, F), jnp.float32),
    )(x, w, b2d)


def _tc_stats_body(h_ref, h0_ref, c0_ref, c1_ref, o_ref):
    i = pl.program_id(0)

    @pl.when(i == 0)
    def _():
        o_ref[...] = jnp.zeros((8, 256), jnp.float32)

    rowid = (lax.broadcasted_iota(jnp.int32, (BLK, 1), 0) + i * BLK)
    m = (((c0_ref[...] + c1_ref[...]) > 0.0) & (rowid < N)).astype(jnp.float32)
    hh = jnp.concatenate([h_ref[...], h0_ref[...]], axis=1)
    mh = hh * m
    o_ref[0:1, :] += jnp.sum(mh, axis=0, keepdims=True)
    o_ref[1:2, :] += jnp.sum(mh * hh, axis=0, keepdims=True)
    o_ref[2:3, :] += jnp.sum(m) * jnp.ones((1, 256), jnp.float32)


def _tc_stats(h, h0, c0, c1):
    return pl.pallas_call(
        _tc_stats_body,
        grid=(NP // BLK,),
        in_specs=[_row_spec(BLK), _row_spec(BLK), _col_spec(BLK), _col_spec(BLK)],
        out_specs=_fix((8, 256)),
        out_shape=jax.ShapeDtypeStruct((8, 256), jnp.float32),
    )(h, h0, c0, c1)


def _tc_bnmm_body(h_ref, h0_ref, st_ref, g_ref, be_ref, tw_ref, tb_ref, w1_ref,
                  o_ref):
    cnt = jnp.maximum(st_ref[2:3, :], 1.0)
    mean = st_ref[0:1, :] / cnt
    var = jnp.maximum(st_ref[1:2, :] / cnt - mean * mean, 0.0)
    g = g_ref[...]
    g2 = g * g
    scale = g2 * lax.rsqrt(g2 * var + EPS * (var + EPS))
    shift = be_ref[...] - scale * mean
    hh = jnp.concatenate([h_ref[...], h0_ref[...]], axis=1)
    z = hh * scale + shift
    hs = jnp.tanh(
        jnp.dot(z, tw_ref[...], preferred_element_type=jnp.float32) + tb_ref[...]
    )
    o_ref[...] = jnp.dot(hs, w1_ref[...], preferred_element_type=jnp.float32)


def _tc_bnmm(h, h0, stats, gam, bet, tw, tb, w1):
    return pl.pallas_call(
        _tc_bnmm_body,
        grid=(NP // BLK,),
        in_specs=[
            _row_spec(BLK), _row_spec(BLK), _fix((8, 256)), _fix((1, 256)),
            _fix((1, 256)), _fix((256, F)), _fix((1, F)), _fix((F, F)),
        ],
        out_specs=_row_spec(BLK),
        out_shape=jax.ShapeDtypeStruct((NP, F), jnp.float32),
    )(h, h0, stats, gam, bet, tw, tb, w1)


def _tc_relumm_body(p0_ref, p1_ref, b_ref, w_ref, o_ref):
    x = jnp.maximum(p0_ref[...] + p1_ref[...] + b_ref[...], 0.0)
    o_ref[...] = jnp.dot(x, w_ref[...], preferred_element_type=jnp.float32)


def _tc_relumm(p0, p1, b2d, w):
    return pl.pallas_call(
        _tc_relumm_body,
        grid=(NP // BLK,),
        in_specs=[_row_spec(BLK), _row_spec(BLK), _fix((1, F)), _fix((F, F))],
        out_specs=_row_spec(BLK),
        out_shape=jax.ShapeDtypeStruct((NP, F), jnp.float32),
    )(p0, p1, b2d, w)


def _tc_update_body(q0_ref, q1_ref, b_ref, h_ref, c0_ref, c1_ref, o_ref):
    m = (c0_ref[...] + c1_ref[...]) > 0.0
    hs = jnp.maximum(q0_ref[...] + q1_ref[...] + b_ref[...], 0.0)
    o_ref[...] = jnp.where(m, hs, h_ref[...])


def _tc_update(q0, q1, b2d, h, c0, c1):
    return pl.pallas_call(
        _tc_update_body,
        grid=(NP // BLK,),
        in_specs=[
            _row_spec(BLK), _row_spec(BLK), _fix((1, F)), _row_spec(BLK),
            _col_spec(BLK), _col_spec(BLK),
        ],
        out_specs=_row_spec(BLK),
        out_shape=jax.ShapeDtypeStruct((NP, F), jnp.float32),
    )(q0, q1, b2d, h, c0, c1)


def _tc_upstats_body(q0_ref, q1_ref, b_ref, h_ref, c0_ref, cn_ref, h0_ref,
                    oh_ref, ost_ref):
    i = pl.program_id(0)

    @pl.when(i == 0)
    def _():
        ost_ref[...] = jnp.zeros((8, 256), jnp.float32)

    m = c0_ref[...] > 0.0
    hs = jnp.maximum(q0_ref[...] + q1_ref[...] + b_ref[...], 0.0)
    hnew = jnp.where(m, hs, h_ref[...])
    oh_ref[...] = hnew
    rowid = (lax.broadcasted_iota(jnp.int32, (BLK, 1), 0) + i * BLK)
    mn = ((cn_ref[...] > 0.0) & (rowid < N)).astype(jnp.float32)
    hh = jnp.concatenate([hnew, h0_ref[...]], axis=1)
    mh = hh * mn
    ost_ref[0:1, :] += jnp.sum(mh, axis=0, keepdims=True)
    ost_ref[1:2, :] += jnp.sum(mh * hh, axis=0, keepdims=True)
    ost_ref[2:3, :] += jnp.sum(mn) * jnp.ones((1, 256), jnp.float32)


def _tc_upstats(q0, q1, b2d, h, c0, cn, h0):
    return pl.pallas_call(
        _tc_upstats_body,
        grid=(NP // BLK,),
        in_specs=[
            _row_spec(BLK), _row_spec(BLK), _fix((1, F)), _row_spec(BLK),
            _col_spec(BLK), _col_spec(BLK), _row_spec(BLK),
        ],
        out_specs=[_row_spec(BLK), _fix((8, 256))],
        out_shape=[
            jax.ShapeDtypeStruct((NP, F), jnp.float32),
            jax.ShapeDtypeStruct((8, 256), jnp.float32),
        ],
    )(q0, q1, b2d, h, c0, cn, h0)


def _tc_final_body(h_ref, gid_ref, y_ref, w_ref, b_ref, yp_ref, ls_ref, pooled):
    i = pl.program_id(0)
    nb = pl.num_programs(0)

    @pl.when(i == 0)
    def _():
        pooled[...] = jnp.full((B, F), -1e30, jnp.float32)

    gid = gid_ref[...]
    hb = h_ref[...]
    for b in range(B):
        mb = gid == b
        vb = jnp.where(mb, hb, -1e30)
        pooled[b:b + 1, :] = jnp.maximum(
            pooled[b:b + 1, :], jnp.max(vb, axis=0, keepdims=True)
        )

    @pl.when(i == nb - 1)
    def _():
        logits = jnp.sum(pooled[...] * w_ref[...], axis=1, keepdims=True) + b_ref[0, 0]
        y = y_ref[...]
        ll = (jnp.maximum(logits, 0.0) - logits * y
              + jnp.log1p(jnp.exp(-jnp.abs(logits))))
        ls_ref[...] = jnp.mean(ll) * jnp.ones((8, 128), jnp.float32)
        yp_ref[...] = jnp.broadcast_to(jax.nn.sigmoid(logits), (B, F))


def _tc_final(h, gid2d, y2d, w2d, b2d):
    return pl.pallas_call(
        _tc_final_body,
        grid=(NP // FBLK,),
        in_specs=[
            _row_spec(FBLK), _col_spec(FBLK), _fix((B, 1)), _fix((1, F)),
            _fix((1, 1)),
        ],
        out_specs=[_fix((B, F)), _fix((8, 128))],
        out_shape=[
            jax.ShapeDtypeStruct((B, F), jnp.float32),
            jax.ShapeDtypeStruct((8, 128), jnp.float32),
        ],
        scratch_shapes=[pltpu.VMEM((B, F), jnp.float32)],
    )(h, gid2d, y2d, w2d, b2d)


def kernel(word_ids, edge_index, edge_weight, edge_time, graph_ids, y_data,
           word_embeds, adapt_W, adapt_b, bn_gamma, bn_beta, temp_W, temp_b,
           gcn_W1, gcn_b1, gcn_W2, gcn_b2, out_W, out_b):
    i32 = jnp.int32
    f32 = jnp.float32
    wi = jnp.concatenate(
        [word_ids.astype(i32), jnp.zeros((NE - N,), i32)])
    src = edge_index[0].astype(i32)
    dst = edge_index[1].astype(i32)
    et = edge_time.astype(i32)
    ew = edge_weight.astype(f32)

    onehot = (et[:, None] == jnp.arange(T, dtype=i32)[None, :]).astype(i32)
    cums = jnp.cumsum(onehot, axis=0)
    counts = cums[-1]
    rank = jnp.sum(cums * onehot, axis=1) - 1
    cnt_pad = ((counts + 7) // 8) * 8 + K
    offs = jnp.concatenate([jnp.zeros((1,), i32), jnp.cumsum(cnt_pad)])[:T]
    pos = offs[et] + rank
    src_s = jnp.zeros((EP,), i32).at[pos].set(src)
    dst_s = jnp.zeros((EP,), i32).at[pos].set(dst)
    ew_s = jnp.zeros((EP,), f32).at[pos].set(ew)
    ends = offs + counts
    offends = jnp.concatenate([offs, ends])

    zeros_nf = jnp.zeros((NP, F), f32)
    gid2d = jnp.concatenate(
        [graph_ids.astype(i32), jnp.full((NP - N,), B, i32)]).reshape(NP, 1)

    gam = bn_gamma.astype(f32).reshape(1, 256)
    bet = bn_beta.astype(f32).reshape(1, 256)
    tb2 = temp_b.astype(f32).reshape(1, F)
    ab2 = adapt_b.astype(f32).reshape(1, F)
    b12 = gcn_b1.astype(f32).reshape(1, F)
    b22 = gcn_b2.astype(f32).reshape(1, F)
    y2d = y_data.astype(f32).reshape(B, 1)
    w2d = out_W.astype(f32).reshape(1, F)
    ob2 = out_b.astype(f32).reshape(1, 1)

    h0raw = _sc_embed(word_embeds.astype(f32), wi)
    h0 = _tc_linear(h0raw[:NP], adapt_W.astype(f32), ab2)

    pat = (jnp.arange(F, dtype=i32)[None, :] // 16
           == jnp.arange(T, dtype=i32)[:, None]).astype(f32)
    pat = jnp.broadcast_to(pat[:, None, :], (T, K, F)).reshape(T * K, F)
    cnt_out = _sc_masks(src_s, dst_s, offends, zeros_nf, pat)
    cntr = cnt_out[0] + cnt_out[1]

    h = h0
    zc = jnp.zeros((NP, 1), f32)
    tw = temp_W.astype(f32)
    w1 = gcn_W1.astype(f32)
    w2 = gcn_W2.astype(f32)
    ccol = [cntr[:, 16 * t:16 * t + 1] for t in range(T)]
    stats = _tc_stats(h0, h0, ccol[0], zc)
    for t in range(T):
        g1 = _tc_bnmm(h, h0, stats, gam, bet, tw, tb2, w1)
        bounds = jnp.concatenate(
            [offs[t:t + 1], ends[t:t + 1], jnp.zeros((14,), i32)])
        p = _sc_msg(g1, src_s, dst_s, ew_s, bounds, zeros_nf)
        g2 = _tc_relumm(p[0], p[1], b12, w2)
        q = _sc_msg(g2, src_s, dst_s, ew_s, bounds, zeros_nf)
        if t < T - 1:
            h, stats = _tc_upstats(q[0], q[1], b22, h, ccol[t], ccol[t + 1], h0)
        else:
            h = _tc_update(q[0], q[1], b22, h, ccol[t], zc)

    yp, ls = _tc_final(h, gid2d, y2d, w2d, ob2)
    return ls[0, 0], yp[:, :1]
